# Initial kernel scaffold; baseline (speedup 1.0000x reference)
import jax
import jax.numpy as jnp
from jax.experimental import pallas as pl


def kernel(x):
    raise NotImplementedError("write your pallas kernel here")


# speedup vs baseline: 6.0103x; 6.0103x over previous
import jax
import jax.numpy as jnp
from jax.experimental import pallas as pl
from jax.experimental.pallas import tpu as pltpu

_STEP = 0.125
_MAX_DIM = 128
_N_LABELS = 16 * 16 * 16


def _lab_kernel(x_ref, lab_ref, norm_ref):
    xb = x_ref[...]
    inv = 1.0 / _STEP
    hi = float(int(2.0 / _STEP) - 1)

    def bucket(col):
        return jnp.floor(jnp.minimum(xb[col : col + 1, :] * inv + inv, hi))

    lab = bucket(0) + bucket(1) * (2.0 / _STEP) + bucket(2) * (2.0 / _STEP) ** 2
    lab_ref[...] = lab.astype(jnp.int32)
    nrm = jnp.sqrt(
        xb[3:4, :] ** 2 + xb[4:5, :] ** 2 + xb[5:6, :] ** 2
    )
    norm_ref[...] = nrm


def _pack_kernel(off_ref, m_ref, pts_ref, out_ref, scratch, sem):
    l = pl.program_id(0)
    s = off_ref[l]
    m = m_ref[l]
    cp = pltpu.make_async_copy(pts_ref.at[pl.ds(s, _MAX_DIM)], scratch, sem)
    cp.start()
    cp.wait()
    j = jax.lax.broadcasted_iota(jnp.int32, (_MAX_DIM, 8), 0)
    out_ref[...] = jnp.where(j < m, scratch[...], 0.0)[None]


def kernel(x):
    n = x.shape[0]
    d = x.shape[1]
    blk = 25600
    grid = pl.cdiv(n, blk)
    xt = x.T
    labels, norms = pl.pallas_call(
        _lab_kernel,
        grid=(grid,),
        in_specs=[pl.BlockSpec((d, blk), lambda i: (0, i))],
        out_specs=[
            pl.BlockSpec((1, blk), lambda i: (0, i)),
            pl.BlockSpec((1, blk), lambda i: (0, i)),
        ],
        out_shape=[
            jax.ShapeDtypeStruct((1, n), jnp.int32),
            jax.ShapeDtypeStruct((1, n), jnp.float32),
        ],
    )(xt)
    labels = labels.reshape(n)
    norms = norms.reshape(n)

    idx = jnp.arange(n, dtype=jnp.int32)
    labs_s, _, perm = jax.lax.sort((labels, norms, idx), num_keys=2, is_stable=True)
    pts_s = x[perm]

    bins = jnp.arange(_N_LABELS, dtype=jnp.int32)
    starts = jnp.searchsorted(labs_s, bins, side="left").astype(jnp.int32)
    ends = jnp.searchsorted(labs_s, bins, side="right").astype(jnp.int32)
    m = jnp.minimum(ends - starts, _MAX_DIM)
    off = ends - m

    pts_pad = jnp.zeros((n + _MAX_DIM, 8), jnp.float32).at[:n, :d].set(pts_s)

    out = pl.pallas_call(
        _pack_kernel,
        grid_spec=pltpu.PrefetchScalarGridSpec(
            num_scalar_prefetch=2,
            grid=(_N_LABELS,),
            in_specs=[pl.BlockSpec(memory_space=pl.ANY)],
            out_specs=pl.BlockSpec((1, _MAX_DIM, 8), lambda l, o, mm: (l, 0, 0)),
            scratch_shapes=[
                pltpu.VMEM((_MAX_DIM, 8), jnp.float32),
                pltpu.SemaphoreType.DMA,
            ],
        ),
        out_shape=jax.ShapeDtypeStruct((_N_LABELS, _MAX_DIM, 8), jnp.float32),
    )(off, m, pts_pad)
    return out[:, :, :d]
